# Initial kernel scaffold; baseline (speedup 1.0000x reference)
import jax
import jax.numpy as jnp
from jax.experimental import pallas as pl


def kernel(distance, FEATURE, centers):
    raise NotImplementedError("write your pallas kernel here")


# speedup vs baseline: 2.5921x; 2.5921x over previous
import functools

import jax
import jax.numpy as jnp
from jax import lax
from jax.experimental import pallas as pl
from jax.experimental.pallas import tpu as pltpu
from jax.experimental.pallas import tpu_sc as plsc

E = 320000
N_NODES = 10000
D = 128
BINS = 64
OUT_D = 2 * D + 3 * BINS

NC = 2
NS = 16
NW = NC * NS
PER_W = E // NW
CH = 80
N_CHUNKS = PER_W // CH


def _sc_body(dist_hbm, feat_hbm, cent_hbm, out_hbm,
             dist_v, idx0_v, idx1_v, d_v, r6, r7, rbf, cent_v, gsem):
    wid = lax.axis_index("s") * NC + lax.axis_index("c")
    base = wid * PER_W
    pltpu.sync_copy(cent_hbm, cent_v)
    cvecs = [cent_v[pl.ds(gg * 16, 16)] for gg in range(BINS // 16)]

    def chunk_body(g, carry):
        b0 = base + g * CH
        pltpu.sync_copy(dist_hbm.at[pl.ds(b0 * 3, CH * 3)], dist_v)

        def extract(i, c):
            lin = (lax.iota(jnp.int32, 16) + i * 16) * 3
            f0 = plsc.load_gather(dist_v, [lin])
            f1 = plsc.load_gather(dist_v, [lin + 1])
            f2 = plsc.load_gather(dist_v, [lin + 2])
            idx0_v[pl.ds(i * 16, 16)] = f0.astype(jnp.int32)
            idx1_v[pl.ds(i * 16, 16)] = f1.astype(jnp.int32)
            d_v[pl.ds(i * 16, 16)] = f2
            return c

        lax.fori_loop(0, CH // 16, extract, 0, unroll=True)

        cp6 = pltpu.make_async_copy(feat_hbm.at[idx0_v], r6, gsem)
        cp6.start()
        cp7 = pltpu.make_async_copy(feat_hbm.at[idx1_v], r7, gsem)
        cp7.start()

        def group_body(i, c):
            dvec = d_v[pl.ds(i * 16, 16)]
            for e16 in range(16):
                e = i * 16 + e16
                dv = jnp.full((16,), dvec[e16], jnp.float32)
                for gg in range(BINS // 16):
                    diff = dv - cvecs[gg]
                    t = diff * diff
                    rbf[e, pl.ds(gg * 16, 16)] = jnp.exp(t * -100.0)
                    rbf[e, pl.ds(BINS + gg * 16, 16)] = jnp.exp(t * -10.0)
                    rbf[e, pl.ds(2 * BINS + gg * 16, 16)] = jnp.exp(-t)
            return c

        lax.fori_loop(0, CH // 16, group_body, 0)

        cp6.wait()
        cp7.wait()

        pltpu.sync_copy(r6, out_hbm.at[pl.ds(b0, CH), pl.ds(0, D)])
        pltpu.sync_copy(r7, out_hbm.at[pl.ds(b0, CH), pl.ds(D, D)])
        pltpu.sync_copy(rbf, out_hbm.at[pl.ds(b0, CH), pl.ds(2 * D, 3 * BINS)])
        return carry

    lax.fori_loop(0, N_CHUNKS, chunk_body, 0)


def kernel(distance, FEATURE, centers):
    mesh = plsc.VectorSubcoreMesh(core_axis_name="c", subcore_axis_name="s")
    run = functools.partial(
        pl.kernel,
        out_type=jax.ShapeDtypeStruct((E, OUT_D), jnp.float32),
        mesh=mesh,
        compiler_params=pltpu.CompilerParams(needs_layout_passes=False),
        scratch_types=[
            pltpu.VMEM((CH * 3,), jnp.float32),
            pltpu.VMEM((CH,), jnp.int32),
            pltpu.VMEM((CH,), jnp.int32),
            pltpu.VMEM((CH,), jnp.float32),
            pltpu.VMEM((CH, D), jnp.float32),
            pltpu.VMEM((CH, D), jnp.float32),
            pltpu.VMEM((CH, 3 * BINS), jnp.float32),
            pltpu.VMEM((BINS,), jnp.float32),
            pltpu.SemaphoreType.DMA,
        ],
    )(_sc_body)
    return run(distance.reshape(-1), FEATURE, centers)


# speedup vs baseline: 2.9026x; 1.1198x over previous
import functools

import jax
import jax.numpy as jnp
from jax import lax
from jax.experimental import pallas as pl
from jax.experimental.pallas import tpu as pltpu
from jax.experimental.pallas import tpu_sc as plsc

E = 320000
N_NODES = 10000
D = 128
BINS = 64
OUT_D = 2 * D + 3 * BINS

NC = 2
NS = 16
NW = NC * NS
PER_W = E // NW
CH = 80
N_CHUNKS = PER_W // CH


def _sc_body(dist_hbm, feat_hbm, cent_hbm, out_hbm,
             dist_v, idx0_v, idx1_v, d_v, r6, r7, rbf, cent_v,
             dsem, gsem, wsem):
    wid = lax.axis_index("s") * NC + lax.axis_index("c")
    base = wid * PER_W
    pltpu.sync_copy(cent_hbm, cent_v)
    cvecs = [cent_v[pl.ds(gg * 16, 16)] for gg in range(BINS // 16)]

    def dist_copy(g, p):
        return pltpu.make_async_copy(
            dist_hbm.at[pl.ds((base + g * CH) * 3, CH * 3)], dist_v[p],
            dsem[p])

    def write_copies(g, p):
        b0 = base + g * CH
        return [
            pltpu.make_async_copy(
                r6[p], out_hbm.at[pl.ds(b0, CH), pl.ds(0, D)], wsem[p]),
            pltpu.make_async_copy(
                r7[p], out_hbm.at[pl.ds(b0, CH), pl.ds(D, D)], wsem[p]),
            pltpu.make_async_copy(
                rbf[p], out_hbm.at[pl.ds(b0, CH), pl.ds(2 * D, 3 * BINS)],
                wsem[p]),
        ]

    def build_and_gather(p):
        def extract(i, c):
            lin = (lax.iota(jnp.int32, 16) + i * 16) * 3
            f0 = plsc.load_gather(dist_v[p], [lin])
            f1 = plsc.load_gather(dist_v[p], [lin + 1])
            f2 = plsc.load_gather(dist_v[p], [lin + 2])
            idx0_v[p][pl.ds(i * 16, 16)] = f0.astype(jnp.int32)
            idx1_v[p][pl.ds(i * 16, 16)] = f1.astype(jnp.int32)
            d_v[p][pl.ds(i * 16, 16)] = f2
            return c

        lax.fori_loop(0, CH // 16, extract, 0, unroll=True)
        cp6 = pltpu.make_async_copy(feat_hbm.at[idx0_v[p]], r6[p], gsem[p])
        cp6.start()
        cp7 = pltpu.make_async_copy(feat_hbm.at[idx1_v[p]], r7[p], gsem[p])
        cp7.start()
        return cp6, cp7

    def compute_rbf(p):
        def group_body(i, c):
            dvec = d_v[p][pl.ds(i * 16, 16)]
            for e16 in range(16):
                e = i * 16 + e16
                dv = jnp.full((16,), dvec[e16], jnp.float32)
                for gg in range(BINS // 16):
                    diff = dv - cvecs[gg]
                    t = diff * diff
                    rbf[p][e, pl.ds(gg * 16, 16)] = jnp.exp(t * -100.0)
                    rbf[p][e, pl.ds(BINS + gg * 16, 16)] = jnp.exp(t * -10.0)
                    rbf[p][e, pl.ds(2 * BINS + gg * 16, 16)] = jnp.exp(-t)
            return c

        lax.fori_loop(0, CH // 16, group_body, 0)

    def do_chunk(g, p, wait_writes, prefetch_next):
        if wait_writes:
            for cp in write_copies(g, p):
                cp.wait()
        dist_copy(g, p).wait()
        cp6, cp7 = build_and_gather(p)
        if prefetch_next:
            dist_copy(g + 1, 1 - p).start()
        compute_rbf(p)
        cp6.wait()
        cp7.wait()
        for cp in write_copies(g, p):
            cp.start()

    dist_copy(0, 0).start()
    do_chunk(0, 0, wait_writes=False, prefetch_next=True)
    do_chunk(1, 1, wait_writes=False, prefetch_next=True)

    def pair_body(t, c):
        do_chunk(2 * t, 0, wait_writes=True, prefetch_next=True)
        do_chunk(2 * t + 1, 1, wait_writes=True, prefetch_next=True)
        return c

    lax.fori_loop(1, (N_CHUNKS - 1) // 2, pair_body, 0)

    do_chunk(N_CHUNKS - 1, (N_CHUNKS - 1) % 2, wait_writes=True,
             prefetch_next=False)
    for cp in write_copies(N_CHUNKS - 2, (N_CHUNKS - 2) % 2):
        cp.wait()
    for cp in write_copies(N_CHUNKS - 1, (N_CHUNKS - 1) % 2):
        cp.wait()


def kernel(distance, FEATURE, centers):
    mesh = plsc.VectorSubcoreMesh(core_axis_name="c", subcore_axis_name="s")
    run = functools.partial(
        pl.kernel,
        out_type=jax.ShapeDtypeStruct((E, OUT_D), jnp.float32),
        mesh=mesh,
        compiler_params=pltpu.CompilerParams(needs_layout_passes=False),
        scratch_types=[
            [pltpu.VMEM((CH * 3,), jnp.float32) for _ in range(2)],
            [pltpu.VMEM((CH,), jnp.int32) for _ in range(2)],
            [pltpu.VMEM((CH,), jnp.int32) for _ in range(2)],
            [pltpu.VMEM((CH,), jnp.float32) for _ in range(2)],
            [pltpu.VMEM((CH, D), jnp.float32) for _ in range(2)],
            [pltpu.VMEM((CH, D), jnp.float32) for _ in range(2)],
            [pltpu.VMEM((CH, 3 * BINS), jnp.float32) for _ in range(2)],
            pltpu.VMEM((BINS,), jnp.float32),
            [pltpu.SemaphoreType.DMA for _ in range(2)],
            [pltpu.SemaphoreType.DMA for _ in range(2)],
            [pltpu.SemaphoreType.DMA for _ in range(2)],
        ],
    )(_sc_body)
    return run(distance.reshape(-1), FEATURE, centers)


# speedup vs baseline: 3.3914x; 1.1684x over previous
import functools

import jax
import jax.numpy as jnp
from jax import lax
from jax.experimental import pallas as pl
from jax.experimental.pallas import tpu as pltpu
from jax.experimental.pallas import tpu_sc as plsc

E = 320000
N_NODES = 10000
D = 128
BINS = 64
OUT_D = 2 * D + 3 * BINS

NC = 2
NS = 16
NW = NC * NS
PER_W = E // NW
CH = 80
N_CHUNKS = PER_W // CH


def _sc_body(idx0_hbm, idx1_hbm, d_hbm, feat_hbm, cent_hbm, out_hbm,
             idx0_v, idx1_v, d_v, r6, r7, rbf, cent_v,
             dsem, gsem, wsem):
    wid = lax.axis_index("s") * NC + lax.axis_index("c")
    base = wid * PER_W
    pltpu.sync_copy(cent_hbm, cent_v)
    cvecs = [cent_v[pl.ds(gg * 16, 16)] for gg in range(BINS // 16)]

    def in_copies(g, p):
        sl = pl.ds(base + g * CH, CH)
        return [
            pltpu.make_async_copy(idx0_hbm.at[sl], idx0_v[p], dsem[p]),
            pltpu.make_async_copy(idx1_hbm.at[sl], idx1_v[p], dsem[p]),
            pltpu.make_async_copy(d_hbm.at[sl], d_v[p], dsem[p]),
        ]

    def write_copies(g, p):
        b0 = base + g * CH
        return [
            pltpu.make_async_copy(
                r6[p], out_hbm.at[pl.ds(b0, CH), pl.ds(0, D)], wsem[p]),
            pltpu.make_async_copy(
                r7[p], out_hbm.at[pl.ds(b0, CH), pl.ds(D, D)], wsem[p]),
            pltpu.make_async_copy(
                rbf[p], out_hbm.at[pl.ds(b0, CH), pl.ds(2 * D, 3 * BINS)],
                wsem[p]),
        ]

    def fire_gathers(p):
        cp6 = pltpu.make_async_copy(feat_hbm.at[idx0_v[p]], r6[p], gsem[p])
        cp6.start()
        cp7 = pltpu.make_async_copy(feat_hbm.at[idx1_v[p]], r7[p], gsem[p])
        cp7.start()
        return cp6, cp7

    def compute_rbf(p):
        def group_body(i, c):
            dvec = d_v[p][pl.ds(i * 16, 16)]
            for e16 in range(16):
                e = i * 16 + e16
                dv = jnp.full((16,), dvec[e16], jnp.float32)
                for gg in range(BINS // 16):
                    diff = dv - cvecs[gg]
                    t = diff * diff
                    rbf[p][e, pl.ds(gg * 16, 16)] = jnp.exp(t * -100.0)
                    rbf[p][e, pl.ds(BINS + gg * 16, 16)] = jnp.exp(t * -10.0)
                    rbf[p][e, pl.ds(2 * BINS + gg * 16, 16)] = jnp.exp(-t)
            return c

        lax.fori_loop(0, CH // 16, group_body, 0)

    def do_chunk(g, p, wait_writes, prefetch_next):
        if wait_writes:
            for cp in write_copies(g, p):
                cp.wait()
        for cp in in_copies(g, p):
            cp.wait()
        cp6, cp7 = fire_gathers(p)
        if prefetch_next:
            for cp in in_copies(g + 1, 1 - p):
                cp.start()
        compute_rbf(p)
        cp6.wait()
        cp7.wait()
        for cp in write_copies(g, p):
            cp.start()

    for cp in in_copies(0, 0):
        cp.start()
    do_chunk(0, 0, wait_writes=False, prefetch_next=True)
    do_chunk(1, 1, wait_writes=False, prefetch_next=True)

    def pair_body(t, c):
        do_chunk(2 * t, 0, wait_writes=True, prefetch_next=True)
        do_chunk(2 * t + 1, 1, wait_writes=True, prefetch_next=True)
        return c

    lax.fori_loop(1, (N_CHUNKS - 1) // 2, pair_body, 0)

    do_chunk(N_CHUNKS - 1, (N_CHUNKS - 1) % 2, wait_writes=True,
             prefetch_next=False)
    for cp in write_copies(N_CHUNKS - 2, (N_CHUNKS - 2) % 2):
        cp.wait()
    for cp in write_copies(N_CHUNKS - 1, (N_CHUNKS - 1) % 2):
        cp.wait()


def kernel(distance, FEATURE, centers):
    mesh = plsc.VectorSubcoreMesh(core_axis_name="c", subcore_axis_name="s")
    run = functools.partial(
        pl.kernel,
        out_type=jax.ShapeDtypeStruct((E, OUT_D), jnp.float32),
        mesh=mesh,
        compiler_params=pltpu.CompilerParams(needs_layout_passes=False),
        scratch_types=[
            [pltpu.VMEM((CH,), jnp.int32) for _ in range(2)],
            [pltpu.VMEM((CH,), jnp.int32) for _ in range(2)],
            [pltpu.VMEM((CH,), jnp.float32) for _ in range(2)],
            [pltpu.VMEM((CH, D), jnp.float32) for _ in range(2)],
            [pltpu.VMEM((CH, D), jnp.float32) for _ in range(2)],
            [pltpu.VMEM((CH, 3 * BINS), jnp.float32) for _ in range(2)],
            pltpu.VMEM((BINS,), jnp.float32),
            [pltpu.SemaphoreType.DMA for _ in range(2)],
            [pltpu.SemaphoreType.DMA for _ in range(2)],
            [pltpu.SemaphoreType.DMA for _ in range(2)],
        ],
    )(_sc_body)
    idx0 = distance[:, 0].astype(jnp.int32)
    idx1 = distance[:, 1].astype(jnp.int32)
    d = distance[:, 2]
    return run(idx0, idx1, d, FEATURE, centers)
